# baseline (device time: 20825 ns/iter reference)
import os

import jax
import jax.numpy as jnp
from jax import lax
from jax.experimental import pallas as pl
from jax.experimental.pallas import tpu as pltpu

_VARIANT = os.environ.get("KVARIANT", "full")
_F32MM = "f32mm" in _VARIANT

_N_CHUNKS = 4


def kernel(x, router, W1, W2):
    t_loc, d = x.shape
    e_loc, _, f = W1.shape
    t_ck = t_loc // _N_CHUNKS

    def body(x_ref, rt_ref, w1_ref, w2_ref, out_ref,
             xsend_ref, xpeer_ref, rtpeer_ref, wsend_ref, wrecv_ref,
             cb_ref, partial_ref, send_sems, recv_sems):
        my_x = lax.axis_index("x")
        my_y = lax.axis_index("y")
        my_z = lax.axis_index("z")
        peer = (1 - my_x, my_y, my_z)
        mesh_t = pl.DeviceIdType.MESH

        barrier_sem = pltpu.get_barrier_semaphore()
        pl.semaphore_signal(barrier_sem, inc=1, device_id=peer,
                            device_id_type=mesh_t)
        pl.semaphore_wait(barrier_sem, 1)

        rdma_r = pltpu.make_async_remote_copy(
            src_ref=rt_ref, dst_ref=rtpeer_ref,
            send_sem=send_sems.at[1], recv_sem=recv_sems.at[1],
            device_id=peer, device_id_type=mesh_t)
        rdma_r.start()
        xsend_ref[...] = x_ref[...].astype(jnp.bfloat16)
        rdma_x = pltpu.make_async_remote_copy(
            src_ref=xsend_ref, dst_ref=xpeer_ref,
            send_sem=send_sems.at[0], recv_sem=recv_sems.at[0],
            device_id=peer, device_id_type=mesh_t)
        rdma_x.start()
        rdma_r.wait()

        dn = (((1,), (1,)), ((), ()))
        gates = jnp.concatenate(
            [lax.dot_general(x_ref[...], rt_ref[...], dn,
                             preferred_element_type=jnp.float32),
             lax.dot_general(x_ref[...], rtpeer_ref[...], dn,
                             preferred_element_type=jnp.float32)],
            axis=1)
        eidx = lax.broadcasted_iota(jnp.int32, (t_loc, 4), 1)
        m1 = jnp.max(gates, axis=1, keepdims=True)
        i1 = jnp.min(jnp.where(gates == m1, eidx, 4), axis=1, keepdims=True)
        masked = jnp.where(eidx == i1, -jnp.inf, gates)
        m2 = jnp.max(masked, axis=1, keepdims=True)
        i2 = jnp.min(jnp.where(masked == m2, eidx, 4), axis=1, keepdims=True)
        b = jnp.exp(m2 - m1)
        w_top = 1.0 / (1.0 + b)
        w_sec = b / (1.0 + b)

        def wcol(c):
            return (jnp.where(i1 == c, w_top, 0.0)
                    + jnp.where(i2 == c, w_sec, 0.0))

        wsend_ref[...] = jnp.concatenate(
            [wcol(2), wcol(3)], axis=1).astype(jnp.bfloat16)
        rdma_w = pltpu.make_async_remote_copy(
            src_ref=wsend_ref, dst_ref=wrecv_ref,
            send_sem=send_sems.at[2], recv_sem=recv_sems.at[2],
            device_id=peer, device_id_type=mesh_t)
        rdma_w.start()

        if _F32MM:
            w1c = [w1_ref[j] for j in range(e_loc)]
            w2c = [w2_ref[j] for j in range(e_loc)]
        else:
            w1c = [w1_ref[j].astype(jnp.bfloat16) for j in range(e_loc)]
            w2c = [w2_ref[j].astype(jnp.bfloat16) for j in range(e_loc)]

        def expert(xin, j):
            h = jnp.maximum(
                jnp.dot(xin, w1c[j], preferred_element_type=jnp.float32), 0.0)
            if not _F32MM:
                h = h.astype(jnp.bfloat16)
            return jnp.dot(h, w2c[j], preferred_element_type=jnp.float32)

        xa = x_ref[...] if _F32MM else xsend_ref[...]

        acc_a = wcol(0) * expert(xa, 0)

        rdma_x.wait()

        rdma_cs = []
        for ck in range(_N_CHUNKS):
            rows = pl.ds(ck * t_ck, t_ck)
            xp = xpeer_ref[rows, :]
            if _F32MM:
                xp = xp.astype(jnp.float32)
            u0 = expert(xp, 0)
            u1 = expert(xp, 1)
            if ck == 0:
                rdma_w.wait()
            wr = wrecv_ref[rows, :].astype(jnp.float32)
            acc_b = wr[:, 0:1] * u0 + wr[:, 1:2] * u1
            cb_ref[rows, :] = acc_b.astype(jnp.bfloat16)
            rdma_c = pltpu.make_async_remote_copy(
                src_ref=cb_ref.at[rows, :], dst_ref=partial_ref.at[rows, :],
                send_sem=send_sems.at[3 + ck], recv_sem=recv_sems.at[3 + ck],
                device_id=peer, device_id_type=mesh_t)
            rdma_c.start()
            rdma_cs.append(rdma_c)

        acc_a = acc_a + wcol(1) * expert(xa, 1)

        for rdma_c in rdma_cs:
            rdma_c.wait()
        out_ref[...] = acc_a + partial_ref[...].astype(jnp.float32)

    return pl.pallas_call(
        body,
        out_shape=jax.ShapeDtypeStruct((t_loc, d), jnp.float32),
        in_specs=[pl.BlockSpec(memory_space=pltpu.VMEM)] * 4,
        out_specs=pl.BlockSpec(memory_space=pltpu.VMEM),
        scratch_shapes=[
            pltpu.VMEM((t_loc, d), jnp.bfloat16),
            pltpu.VMEM((t_loc, d), jnp.bfloat16),
            pltpu.VMEM((e_loc, d), jnp.float32),
            pltpu.VMEM((t_loc, e_loc), jnp.bfloat16),
            pltpu.VMEM((t_loc, e_loc), jnp.bfloat16),
            pltpu.VMEM((t_loc, d), jnp.bfloat16),
            pltpu.VMEM((t_loc, d), jnp.bfloat16),
            pltpu.SemaphoreType.DMA((3 + _N_CHUNKS,)),
            pltpu.SemaphoreType.DMA((3 + _N_CHUNKS,)),
        ],
        compiler_params=pltpu.CompilerParams(collective_id=0),
    )(x, router.T, W1, W2)


# device time: 19762 ns/iter; 1.0538x vs baseline; 1.0538x over previous
import os

import jax
import jax.numpy as jnp
from jax import lax
from jax.experimental import pallas as pl
from jax.experimental.pallas import tpu as pltpu

_VARIANT = os.environ.get("KVARIANT", "full")
_F32MM = "f32mm" in _VARIANT

_N_CHUNKS = 4


def kernel(x, router, W1, W2):
    t_loc, d = x.shape
    e_loc, _, f = W1.shape
    t_ck = t_loc // _N_CHUNKS

    def body(x_ref, rt_ref, w1_ref, w2_ref, out_ref,
             xsend_ref, xpeer_ref, rtpeer_ref, wsend_ref, wrecv_ref,
             cb_ref, partial_ref, send_sems, recv_sems):
        my_x = lax.axis_index("x")
        my_y = lax.axis_index("y")
        my_z = lax.axis_index("z")
        peer = (1 - my_x, my_y, my_z)
        mesh_t = pl.DeviceIdType.MESH

        barrier_sem = pltpu.get_barrier_semaphore()
        pl.semaphore_signal(barrier_sem, inc=1, device_id=peer,
                            device_id_type=mesh_t)
        pl.semaphore_wait(barrier_sem, 1)

        rdma_r = pltpu.make_async_remote_copy(
            src_ref=rt_ref, dst_ref=rtpeer_ref,
            send_sem=send_sems.at[2], recv_sem=recv_sems.at[2],
            device_id=peer, device_id_type=mesh_t)
        rdma_r.start()
        t_half = t_loc // 2
        rdma_xs = []
        for hk in range(2):
            rows = pl.ds(hk * t_half, t_half)
            xsend_ref[rows, :] = x_ref[rows, :].astype(jnp.bfloat16)
            rdma_x = pltpu.make_async_remote_copy(
                src_ref=xsend_ref.at[rows, :], dst_ref=xpeer_ref.at[rows, :],
                send_sem=send_sems.at[hk], recv_sem=recv_sems.at[hk],
                device_id=peer, device_id_type=mesh_t)
            rdma_x.start()
            rdma_xs.append(rdma_x)
        rdma_r.wait()

        dn = (((1,), (1,)), ((), ()))
        gates = jnp.concatenate(
            [lax.dot_general(x_ref[...], rt_ref[...], dn,
                             preferred_element_type=jnp.float32),
             lax.dot_general(x_ref[...], rtpeer_ref[...], dn,
                             preferred_element_type=jnp.float32)],
            axis=1)
        eidx = lax.broadcasted_iota(jnp.int32, (t_loc, 4), 1)
        m1 = jnp.max(gates, axis=1, keepdims=True)
        i1 = jnp.min(jnp.where(gates == m1, eidx, 4), axis=1, keepdims=True)
        masked = jnp.where(eidx == i1, -jnp.inf, gates)
        m2 = jnp.max(masked, axis=1, keepdims=True)
        i2 = jnp.min(jnp.where(masked == m2, eidx, 4), axis=1, keepdims=True)
        b = jnp.exp(m2 - m1)
        w_top = 1.0 / (1.0 + b)
        w_sec = b / (1.0 + b)

        def wcol(c):
            return (jnp.where(i1 == c, w_top, 0.0)
                    + jnp.where(i2 == c, w_sec, 0.0))

        wsend_ref[...] = jnp.concatenate(
            [wcol(2), wcol(3)], axis=1).astype(jnp.bfloat16)
        rdma_w = pltpu.make_async_remote_copy(
            src_ref=wsend_ref, dst_ref=wrecv_ref,
            send_sem=send_sems.at[3], recv_sem=recv_sems.at[3],
            device_id=peer, device_id_type=mesh_t)
        rdma_w.start()

        if _F32MM:
            w1c = [w1_ref[j] for j in range(e_loc)]
            w2c = [w2_ref[j] for j in range(e_loc)]
        else:
            w1c = [w1_ref[j].astype(jnp.bfloat16) for j in range(e_loc)]
            w2c = [w2_ref[j].astype(jnp.bfloat16) for j in range(e_loc)]

        def expert(xin, j):
            h = jnp.maximum(
                jnp.dot(xin, w1c[j], preferred_element_type=jnp.float32), 0.0)
            if not _F32MM:
                h = h.astype(jnp.bfloat16)
            return jnp.dot(h, w2c[j], preferred_element_type=jnp.float32)

        xa = x_ref[...] if _F32MM else xsend_ref[...]

        acc_a = wcol(0) * expert(xa, 0)

        ck_per_half = _N_CHUNKS // 2
        rdma_cs = []
        for ck in range(_N_CHUNKS):
            if ck % ck_per_half == 0:
                rdma_xs[ck // ck_per_half].wait()
            rows = pl.ds(ck * t_ck, t_ck)
            xp = xpeer_ref[rows, :]
            if _F32MM:
                xp = xp.astype(jnp.float32)
            u0 = expert(xp, 0)
            u1 = expert(xp, 1)
            if ck == 0:
                rdma_w.wait()
            wr = wrecv_ref[rows, :].astype(jnp.float32)
            acc_b = wr[:, 0:1] * u0 + wr[:, 1:2] * u1
            cb_ref[rows, :] = acc_b.astype(jnp.bfloat16)
            rdma_c = pltpu.make_async_remote_copy(
                src_ref=cb_ref.at[rows, :], dst_ref=partial_ref.at[rows, :],
                send_sem=send_sems.at[4 + ck], recv_sem=recv_sems.at[4 + ck],
                device_id=peer, device_id_type=mesh_t)
            rdma_c.start()
            rdma_cs.append(rdma_c)

        acc_a = acc_a + wcol(1) * expert(xa, 1)

        for rdma_c in rdma_cs:
            rdma_c.wait()
        out_ref[...] = acc_a + partial_ref[...].astype(jnp.float32)

    return pl.pallas_call(
        body,
        out_shape=jax.ShapeDtypeStruct((t_loc, d), jnp.float32),
        in_specs=[pl.BlockSpec(memory_space=pltpu.VMEM)] * 4,
        out_specs=pl.BlockSpec(memory_space=pltpu.VMEM),
        scratch_shapes=[
            pltpu.VMEM((t_loc, d), jnp.bfloat16),
            pltpu.VMEM((t_loc, d), jnp.bfloat16),
            pltpu.VMEM((e_loc, d), jnp.float32),
            pltpu.VMEM((t_loc, e_loc), jnp.bfloat16),
            pltpu.VMEM((t_loc, e_loc), jnp.bfloat16),
            pltpu.VMEM((t_loc, d), jnp.bfloat16),
            pltpu.VMEM((t_loc, d), jnp.bfloat16),
            pltpu.SemaphoreType.DMA((4 + _N_CHUNKS,)),
            pltpu.SemaphoreType.DMA((4 + _N_CHUNKS,)),
        ],
        compiler_params=pltpu.CompilerParams(collective_id=0),
    )(x, router.T, W1, W2)
